# baseline (speedup 1.0000x reference)
import functools

import jax
import jax.numpy as jnp
from jax.experimental import pallas as pl
from jax.experimental.pallas import tpu as pltpu


def _pick_tile(n, candidates):
    for t in candidates:
        if n % t == 0:
            return t
    return 128


def _stage1_kernel(adj_ref, x0_ref, feats_ref, w_ref, b_ref,
                   x1_ref, v_ref, t_ref, acc_ref, *,
                   n_adj, n_feat, tk_adj, tk_feat, emb):
    k = pl.program_id(1)

    @pl.when(k == 0)
    def _():
        x1_ref[...] = jnp.zeros_like(x1_ref)
        acc_ref[...] = jnp.zeros_like(acc_ref)

    @pl.when(k < n_adj)
    def _():
        x1_ref[...] += jnp.dot(adj_ref[...],
                               x0_ref[pl.ds(k * tk_adj, tk_adj), :],
                               preferred_element_type=jnp.float32)

    @pl.when(k >= n_adj)
    def _():
        kf = k - n_adj
        acc_ref[...] += jnp.dot(feats_ref[...],
                                w_ref[pl.ds(kf * tk_feat, tk_feat), :],
                                preferred_element_type=jnp.float32)

    @pl.when(k == n_adj + n_feat - 1)
    def _():
        proj = acc_ref[...] + b_ref[...]
        v_ref[...] = proj[:, :emb]
        t_ref[...] = proj[:, emb:]


def _stage2_kernel(adj_ref, x1_ref, o_ref, *, tk):
    k = pl.program_id(1)

    @pl.when(k == 0)
    def _():
        o_ref[...] = jnp.zeros_like(o_ref)

    o_ref[...] += jnp.dot(adj_ref[...],
                          x1_ref[pl.ds(k * tk, tk), :],
                          preferred_element_type=jnp.float32)


def kernel(mAdj, itemEmbds, featsPadded, wBlk, bCat):
    n, emb = itemEmbds.shape
    k_pad = featsPadded.shape[1]
    out_w = wBlk.shape[1]

    tm = _pick_tile(n, (512, 256, 128))
    tk_adj = _pick_tile(n, (512, 256, 128))
    tk_feat = _pick_tile(k_pad, (640, 512, 384, 256, 128))
    n_adj = n // tk_adj
    n_feat = k_pad // tk_feat

    stage1 = functools.partial(_stage1_kernel, n_adj=n_adj, n_feat=n_feat,
                               tk_adj=tk_adj, tk_feat=tk_feat, emb=emb)
    flops1 = 2 * n * n * emb + 2 * n * k_pad * out_w
    bytes1 = 4 * (n * n + n * emb + n * k_pad + k_pad * out_w
                  + out_w + n * (emb + out_w))
    last_adj = n_adj - 1

    x1, v, t = pl.pallas_call(
        stage1,
        out_shape=[jax.ShapeDtypeStruct((n, emb), jnp.float32),
                   jax.ShapeDtypeStruct((n, emb), jnp.float32),
                   jax.ShapeDtypeStruct((n, emb), jnp.float32)],
        grid_spec=pltpu.PrefetchScalarGridSpec(
            num_scalar_prefetch=0,
            grid=(n // tm, n_adj + n_feat),
            in_specs=[
                pl.BlockSpec((tm, tk_adj),
                             lambda i, k: (i, jnp.minimum(k, last_adj))),
                pl.BlockSpec((n, emb), lambda i, k: (0, 0)),
                pl.BlockSpec((tm, tk_feat),
                             lambda i, k: (i, jnp.maximum(k - n_adj, 0))),
                pl.BlockSpec((k_pad, out_w), lambda i, k: (0, 0)),
                pl.BlockSpec((1, out_w), lambda i, k: (0, 0)),
            ],
            out_specs=[pl.BlockSpec((tm, emb), lambda i, k: (i, 0)),
                       pl.BlockSpec((tm, emb), lambda i, k: (i, 0)),
                       pl.BlockSpec((tm, emb), lambda i, k: (i, 0))],
            scratch_shapes=[pltpu.VMEM((tm, out_w), jnp.float32)]),
        compiler_params=pltpu.CompilerParams(
            dimension_semantics=("parallel", "arbitrary")),
        cost_estimate=pl.CostEstimate(flops=flops1, transcendentals=0,
                                      bytes_accessed=bytes1),
    )(mAdj, itemEmbds, featsPadded, wBlk, bCat)

    stage2 = functools.partial(_stage2_kernel, tk=tk_adj)
    flops2 = 2 * n * n * emb
    bytes2 = 4 * (n * n + 2 * n * emb)

    items = pl.pallas_call(
        stage2,
        out_shape=jax.ShapeDtypeStruct((n, emb), jnp.float32),
        grid_spec=pltpu.PrefetchScalarGridSpec(
            num_scalar_prefetch=0,
            grid=(n // tm, n_adj),
            in_specs=[pl.BlockSpec((tm, tk_adj), lambda i, k: (i, k)),
                      pl.BlockSpec((n, emb), lambda i, k: (0, 0))],
            out_specs=pl.BlockSpec((tm, emb), lambda i, k: (i, 0))),
        compiler_params=pltpu.CompilerParams(
            dimension_semantics=("parallel", "arbitrary")),
        cost_estimate=pl.CostEstimate(flops=flops2, transcendentals=0,
                                      bytes_accessed=bytes2),
    )(mAdj, x1)

    return items, v, t


# speedup vs baseline: 1.1820x; 1.1820x over previous
import functools

import jax
import jax.numpy as jnp
from jax.experimental import pallas as pl
from jax.experimental.pallas import tpu as pltpu


def _pick_tile(n, candidates):
    for t in candidates:
        if n % t == 0:
            return t
    return 128


def _propagate_kernel(adj_ref, x_ref, o_ref, *, tk):
    k = pl.program_id(1)

    @pl.when(k == 0)
    def _():
        o_ref[...] = jnp.zeros_like(o_ref)

    o_ref[...] += jnp.dot(adj_ref[...],
                          x_ref[pl.ds(k * tk, tk), :],
                          preferred_element_type=jnp.float32)


def _projector_kernel(feats_ref, w_ref, b_ref, v_ref, t_ref, *, emb):
    proj = jnp.dot(feats_ref[...], w_ref[...],
                   preferred_element_type=jnp.float32) + b_ref[...]
    v_ref[...] = proj[:, :emb]
    t_ref[...] = proj[:, emb:]


def kernel(mAdj, itemEmbds, featsPadded, wBlk, bCat):
    n, emb = itemEmbds.shape
    k_pad = featsPadded.shape[1]
    out_w = wBlk.shape[1]

    tm = _pick_tile(n, (512, 256, 128))
    tk = _pick_tile(n, (512, 256, 128))
    n_k = n // tk

    prop = pl.pallas_call(
        functools.partial(_propagate_kernel, tk=tk),
        out_shape=jax.ShapeDtypeStruct((n, emb), jnp.float32),
        grid_spec=pltpu.PrefetchScalarGridSpec(
            num_scalar_prefetch=0,
            grid=(n // tm, n_k),
            in_specs=[pl.BlockSpec((tm, tk), lambda i, k: (i, k)),
                      pl.BlockSpec((n, emb), lambda i, k: (0, 0))],
            out_specs=pl.BlockSpec((tm, emb), lambda i, k: (i, 0))),
        compiler_params=pltpu.CompilerParams(
            dimension_semantics=("parallel", "arbitrary")),
        cost_estimate=pl.CostEstimate(
            flops=2 * n * n * emb, transcendentals=0,
            bytes_accessed=4 * (n * n + 2 * n * emb)),
    )

    x1 = prop(mAdj, itemEmbds)
    items = prop(mAdj, x1)

    tmp = _pick_tile(n, (512, 256, 128))
    v, t = pl.pallas_call(
        functools.partial(_projector_kernel, emb=emb),
        out_shape=[jax.ShapeDtypeStruct((n, emb), jnp.float32),
                   jax.ShapeDtypeStruct((n, emb), jnp.float32)],
        grid_spec=pltpu.PrefetchScalarGridSpec(
            num_scalar_prefetch=0,
            grid=(n // tmp,),
            in_specs=[pl.BlockSpec((tmp, k_pad), lambda i: (i, 0)),
                      pl.BlockSpec((k_pad, out_w), lambda i: (0, 0)),
                      pl.BlockSpec((1, out_w), lambda i: (0, 0))],
            out_specs=[pl.BlockSpec((tmp, emb), lambda i: (i, 0)),
                       pl.BlockSpec((tmp, emb), lambda i: (i, 0))]),
        compiler_params=pltpu.CompilerParams(
            dimension_semantics=("parallel",)),
        cost_estimate=pl.CostEstimate(
            flops=2 * n * k_pad * out_w, transcendentals=0,
            bytes_accessed=4 * (n * k_pad + k_pad * out_w + n * out_w)),
    )(featsPadded, wBlk, bCat)

    return items, v, t


# speedup vs baseline: 1.8192x; 1.5390x over previous
import functools

import jax
import jax.numpy as jnp
from jax.experimental import pallas as pl
from jax.experimental.pallas import tpu as pltpu


def _pick_tile(n, candidates):
    for t in candidates:
        if n % t == 0:
            return t
    return 128


def _propagate_kernel(adj_ref, x_ref, o_ref, *, tk):
    k = pl.program_id(1)

    @pl.when(k == 0)
    def _():
        o_ref[...] = jnp.zeros_like(o_ref)

    o_ref[...] += jnp.dot(adj_ref[...],
                          x_ref[pl.ds(k * tk, tk), :],
                          preferred_element_type=jnp.float32)


def _projector_kernel(feats_ref, w_ref, b_ref, v_ref, t_ref, *, emb):
    proj = jnp.dot(feats_ref[...], w_ref[...],
                   preferred_element_type=jnp.float32) + b_ref[...]
    v_ref[...] = proj[:, :emb]
    t_ref[...] = proj[:, emb:]


def kernel(mAdj, itemEmbds, featsPadded, wBlk, bCat):
    n, emb = itemEmbds.shape
    k_pad = featsPadded.shape[1]
    out_w = wBlk.shape[1]

    tm = _pick_tile(n, (1024, 512, 256, 128))
    tk = _pick_tile(n, (1024, 512, 256, 128))
    n_k = n // tk

    prop = pl.pallas_call(
        functools.partial(_propagate_kernel, tk=tk),
        out_shape=jax.ShapeDtypeStruct((n, emb), jnp.float32),
        grid_spec=pltpu.PrefetchScalarGridSpec(
            num_scalar_prefetch=0,
            grid=(n // tm, n_k),
            in_specs=[pl.BlockSpec((tm, tk), lambda i, k: (i, k)),
                      pl.BlockSpec((n, emb), lambda i, k: (0, 0))],
            out_specs=pl.BlockSpec((tm, emb), lambda i, k: (i, 0))),
        compiler_params=pltpu.CompilerParams(
            dimension_semantics=("parallel", "arbitrary")),
        cost_estimate=pl.CostEstimate(
            flops=2 * n * n * emb, transcendentals=0,
            bytes_accessed=4 * (n * n + 2 * n * emb)),
    )

    x1 = prop(mAdj, itemEmbds)
    items = prop(mAdj, x1)

    tmp = _pick_tile(n, (256, 128))
    v, t = pl.pallas_call(
        functools.partial(_projector_kernel, emb=emb),
        out_shape=[jax.ShapeDtypeStruct((n, emb), jnp.float32),
                   jax.ShapeDtypeStruct((n, emb), jnp.float32)],
        grid_spec=pltpu.PrefetchScalarGridSpec(
            num_scalar_prefetch=0,
            grid=(n // tmp,),
            in_specs=[pl.BlockSpec((tmp, k_pad), lambda i: (i, 0)),
                      pl.BlockSpec((k_pad, out_w), lambda i: (0, 0)),
                      pl.BlockSpec((1, out_w), lambda i: (0, 0))],
            out_specs=[pl.BlockSpec((tmp, emb), lambda i: (i, 0)),
                       pl.BlockSpec((tmp, emb), lambda i: (i, 0))]),
        compiler_params=pltpu.CompilerParams(
            dimension_semantics=("parallel",)),
        cost_estimate=pl.CostEstimate(
            flops=2 * n * k_pad * out_w, transcendentals=0,
            bytes_accessed=4 * (n * k_pad + k_pad * out_w + n * out_w)),
    )(featsPadded, wBlk, bCat)

    return items, v, t


# speedup vs baseline: 1.9647x; 1.0800x over previous
import functools

import jax
import jax.numpy as jnp
from jax.experimental import pallas as pl
from jax.experimental.pallas import tpu as pltpu


def _pick_tile(n, candidates):
    for t in candidates:
        if n % t == 0:
            return t
    return 128


def _propagate_kernel(adj_ref, x_ref, o_ref):
    o_ref[...] = jnp.dot(adj_ref[...], x_ref[...],
                         preferred_element_type=jnp.float32)


def _projector_kernel(feats_ref, w_ref, b_ref, v_ref, t_ref, *, emb):
    proj = jnp.dot(feats_ref[...], w_ref[...],
                   preferred_element_type=jnp.float32) + b_ref[...]
    v_ref[...] = proj[:, :emb]
    t_ref[...] = proj[:, emb:]


def kernel(mAdj, itemEmbds, featsPadded, wBlk, bCat):
    n, emb = itemEmbds.shape
    k_pad = featsPadded.shape[1]
    out_w = wBlk.shape[1]

    tm = _pick_tile(n, (512, 256, 128))

    prop = pl.pallas_call(
        _propagate_kernel,
        out_shape=jax.ShapeDtypeStruct((n, emb), jnp.float32),
        grid_spec=pltpu.PrefetchScalarGridSpec(
            num_scalar_prefetch=0,
            grid=(n // tm,),
            in_specs=[pl.BlockSpec((tm, n), lambda i: (i, 0)),
                      pl.BlockSpec((n, emb), lambda i: (0, 0))],
            out_specs=pl.BlockSpec((tm, emb), lambda i: (i, 0))),
        compiler_params=pltpu.CompilerParams(
            dimension_semantics=("parallel",)),
        cost_estimate=pl.CostEstimate(
            flops=2 * n * n * emb, transcendentals=0,
            bytes_accessed=4 * (n * n + 2 * n * emb)),
    )

    x1 = prop(mAdj, itemEmbds)
    items = prop(mAdj, x1)

    tmp = _pick_tile(n, (256, 128))
    v, t = pl.pallas_call(
        functools.partial(_projector_kernel, emb=emb),
        out_shape=[jax.ShapeDtypeStruct((n, emb), jnp.float32),
                   jax.ShapeDtypeStruct((n, emb), jnp.float32)],
        grid_spec=pltpu.PrefetchScalarGridSpec(
            num_scalar_prefetch=0,
            grid=(n // tmp,),
            in_specs=[pl.BlockSpec((tmp, k_pad), lambda i: (i, 0)),
                      pl.BlockSpec((k_pad, out_w), lambda i: (0, 0)),
                      pl.BlockSpec((1, out_w), lambda i: (0, 0))],
            out_specs=[pl.BlockSpec((tmp, emb), lambda i: (i, 0)),
                       pl.BlockSpec((tmp, emb), lambda i: (i, 0))]),
        compiler_params=pltpu.CompilerParams(
            dimension_semantics=("parallel",)),
        cost_estimate=pl.CostEstimate(
            flops=2 * n * k_pad * out_w, transcendentals=0,
            bytes_accessed=4 * (n * k_pad + k_pad * out_w + n * out_w)),
    )(featsPadded, wBlk, bCat)

    return items, v, t


# speedup vs baseline: 2.1406x; 1.0896x over previous
import functools

import jax
import jax.numpy as jnp
from jax.experimental import pallas as pl
from jax.experimental.pallas import tpu as pltpu


def _pick_tile(n, candidates):
    for t in candidates:
        if n % t == 0:
            return t
    return 128


def _fused_kernel(adj_ref, x0_ref, feats_ref, w_ref, b_ref,
                  items_ref, v_ref, t_ref, x1_ref, *, tm, emb):
    l = pl.program_id(0)
    i = pl.program_id(1)

    @pl.when(l == 0)
    def _():
        x1_ref[pl.ds(i * tm, tm), :] = jnp.dot(
            adj_ref[...], x0_ref[...], preferred_element_type=jnp.float32)

    @pl.when(l == 1)
    def _():
        items_ref[...] = jnp.dot(adj_ref[...], x1_ref[...],
                                 preferred_element_type=jnp.float32)

    @pl.when(l == 2)
    def _():
        proj = jnp.dot(feats_ref[...], w_ref[...],
                       preferred_element_type=jnp.float32) + b_ref[...]
        v_ref[...] = proj[:, :emb]
        t_ref[...] = proj[:, emb:]


def kernel(mAdj, itemEmbds, featsPadded, wBlk, bCat):
    n, emb = itemEmbds.shape
    k_pad = featsPadded.shape[1]
    out_w = wBlk.shape[1]

    tm = _pick_tile(n, (512, 256, 128))
    n_i = n // tm
    last = n_i - 1

    flops = 2 * (2 * n * n * emb + n * k_pad * out_w)
    bytes_accessed = 4 * (2 * n * n + n * k_pad + n * emb
                          + k_pad * out_w + out_w + 3 * n * emb)

    items, v, t = pl.pallas_call(
        functools.partial(_fused_kernel, tm=tm, emb=emb),
        out_shape=[jax.ShapeDtypeStruct((n, emb), jnp.float32),
                   jax.ShapeDtypeStruct((n, emb), jnp.float32),
                   jax.ShapeDtypeStruct((n, emb), jnp.float32)],
        grid_spec=pltpu.PrefetchScalarGridSpec(
            num_scalar_prefetch=0,
            grid=(3, n_i),
            in_specs=[
                pl.BlockSpec((tm, n),
                             lambda l, i: (jnp.where(l == 2, last, i), 0)),
                pl.BlockSpec((n, emb), lambda l, i: (0, 0)),
                pl.BlockSpec((tm, k_pad),
                             lambda l, i: (jnp.where(l == 2, i, 0), 0)),
                pl.BlockSpec((k_pad, out_w), lambda l, i: (0, 0)),
                pl.BlockSpec((1, out_w), lambda l, i: (0, 0)),
            ],
            out_specs=[
                pl.BlockSpec(
                    (tm, emb),
                    lambda l, i: (jnp.where(l == 0, 0,
                                            jnp.where(l == 1, i, last)), 0)),
                pl.BlockSpec((tm, emb),
                             lambda l, i: (jnp.where(l == 2, i, 0), 0)),
                pl.BlockSpec((tm, emb),
                             lambda l, i: (jnp.where(l == 2, i, 0), 0)),
            ],
            scratch_shapes=[pltpu.VMEM((n, emb), jnp.float32)]),
        compiler_params=pltpu.CompilerParams(
            dimension_semantics=("arbitrary", "arbitrary")),
        cost_estimate=pl.CostEstimate(flops=flops, transcendentals=0,
                                      bytes_accessed=bytes_accessed),
    )(mAdj, itemEmbds, featsPadded, wBlk, bCat)

    return items, v, t


# speedup vs baseline: 2.1950x; 1.0254x over previous
import functools

import jax
import jax.numpy as jnp
from jax.experimental import pallas as pl
from jax.experimental.pallas import tpu as pltpu


def _pick_tile(n, candidates):
    for t in candidates:
        if n % t == 0:
            return t
    return 128


def _fused_kernel(adj_ref, x0_ref, feats_ref, w_ref, b_ref,
                  items_ref, v_ref, t_ref, x1_ref, *, tm, emb):
    l = pl.program_id(0)
    i = pl.program_id(1)

    @pl.when(l == 0)
    def _():
        x1_ref[pl.ds(i * tm, tm), :] = jnp.dot(
            adj_ref[...], x0_ref[...], preferred_element_type=jnp.float32)
        proj = jnp.dot(feats_ref[...], w_ref[...],
                       preferred_element_type=jnp.float32) + b_ref[...]
        v_ref[...] = proj[:, :emb]
        t_ref[...] = proj[:, emb:]

    @pl.when(l == 1)
    def _():
        items_ref[...] = jnp.dot(adj_ref[...], x1_ref[...],
                                 preferred_element_type=jnp.float32)


def kernel(mAdj, itemEmbds, featsPadded, wBlk, bCat):
    n, emb = itemEmbds.shape
    k_pad = featsPadded.shape[1]
    out_w = wBlk.shape[1]

    tm = _pick_tile(n, (512, 256, 128))
    n_i = n // tm
    last = n_i - 1

    flops = 2 * (2 * n * n * emb + n * k_pad * out_w)
    bytes_accessed = 4 * (2 * n * n + n * k_pad + n * emb
                          + k_pad * out_w + out_w + 3 * n * emb)

    items, v, t = pl.pallas_call(
        functools.partial(_fused_kernel, tm=tm, emb=emb),
        out_shape=[jax.ShapeDtypeStruct((n, emb), jnp.float32),
                   jax.ShapeDtypeStruct((n, emb), jnp.float32),
                   jax.ShapeDtypeStruct((n, emb), jnp.float32)],
        grid_spec=pltpu.PrefetchScalarGridSpec(
            num_scalar_prefetch=0,
            grid=(2, n_i),
            in_specs=[
                pl.BlockSpec((tm, n), lambda l, i: (i, 0)),
                pl.BlockSpec((n, emb), lambda l, i: (0, 0)),
                pl.BlockSpec((tm, k_pad),
                             lambda l, i: (jnp.where(l == 0, i, last), 0)),
                pl.BlockSpec((k_pad, out_w), lambda l, i: (0, 0)),
                pl.BlockSpec((1, out_w), lambda l, i: (0, 0)),
            ],
            out_specs=[
                pl.BlockSpec((tm, emb),
                             lambda l, i: (jnp.where(l == 1, i, 0), 0)),
                pl.BlockSpec((tm, emb),
                             lambda l, i: (jnp.where(l == 0, i, last), 0)),
                pl.BlockSpec((tm, emb),
                             lambda l, i: (jnp.where(l == 0, i, last), 0)),
            ],
            scratch_shapes=[pltpu.VMEM((n, emb), jnp.float32)]),
        compiler_params=pltpu.CompilerParams(
            dimension_semantics=("arbitrary", "arbitrary")),
        cost_estimate=pl.CostEstimate(flops=flops, transcendentals=0,
                                      bytes_accessed=bytes_accessed),
    )(mAdj, itemEmbds, featsPadded, wBlk, bCat)

    return items, v, t
